# baseline (speedup 1.0000x reference)
import functools
import jax
import jax.numpy as jnp
from jax import lax
from jax.experimental import pallas as pl
from jax.experimental.pallas import tpu as pltpu
from jax.experimental.pallas import tpu_sc as plsc

D = 2048
K = 8
BOUND = 3.0
MIN_BW = 1e-3
MIN_BH = 1e-3
MIN_D = 1e-3
MIN_L = 0.025

NSUB = 2 * K


def _softmax0(v):
    m = jnp.max(v, axis=0, keepdims=True)
    e = jnp.exp(v - m)
    return e / jnp.sum(e, axis=0, keepdims=True)


def _knot_rows(frac):
    acc = frac[0:1, :]
    cs = [acc]
    for k in range(1, K):
        acc = acc + frac[k : k + 1, :]
        cs.append(acc)
    ones = jnp.ones_like(frac[0:1, :])
    rows = [jnp.full_like(frac[0:1, :], -BOUND)]
    for k in range(K - 1):
        rows.append(2.0 * BOUND * cs[k] - BOUND)
    rows.append(BOUND * ones)
    kn = jnp.concatenate(rows, axis=0)
    lengths = kn[1:, :] - kn[:-1, :]
    return lengths, kn


def _prep_kernel(uw_ref, uh_ref, ud_ref, ul_ref, a_ref, b_ref, g_ref, e_ref, bnd_ref):
    uw = uw_ref[...]
    uh = uh_ref[...]
    ud = ud_ref[0 : K - 1, :]
    ul = ul_ref[...]

    w = MIN_BW + (1.0 - MIN_BW * K) * _softmax0(uw)
    h = MIN_BH + (1.0 - MIN_BH * K) * _softmax0(uh)
    wf, cw = _knot_rows(w)
    hf, ch = _knot_rows(h)

    sp = jnp.maximum(ud, 0.0) + jnp.log(1.0 + jnp.exp(-jnp.abs(ud)))
    dmid = MIN_D + sp
    dend = jnp.full_like(dmid[0:1, :], 1.0 - MIN_D)
    dfull = jnp.concatenate([dend, dmid, dend], axis=0)

    lam = (1.0 - 2.0 * MIN_L) / (1.0 + jnp.exp(-ul)) + MIN_L

    d0 = dfull[:-1, :]
    d1 = dfull[1:, :]
    delta = hf / wf
    wb = jnp.sqrt(d0 / d1)
    wc = (lam * d0 + (1.0 - lam) * wb * d1) / delta
    ya = ch[:-1, :]
    yb = ch[:-1, :] + hf
    yc = ((1.0 - lam) * ya + lam * wb * yb) / ((1.0 - lam) + lam * wb)
    iw = 1.0 / wf
    cwl = cw[:-1, :]
    t0 = -cwl * iw
    wcyc = wc * yc
    wbyb = wb * yb

    a1 = ya * lam + t0 * (wcyc - ya)
    b1 = iw * (wcyc - ya)
    g1 = lam + t0 * (wc - 1.0)
    e1 = iw * (wc - 1.0)
    a2 = wcyc - lam * wbyb + t0 * (wbyb - wcyc)
    b2 = iw * (wbyb - wcyc)
    g2 = wc - lam * wb + t0 * (wb - wc)
    e2 = iw * (wb - wc)

    def ilv(p, q):
        rows = []
        for k in range(K):
            rows.append(p[k : k + 1, :])
            rows.append(q[k : k + 1, :])
        return jnp.concatenate(rows, axis=0)

    a_ref[...] = ilv(a1, a2)
    b_ref[...] = ilv(b1, b2)
    g_ref[...] = ilv(g1, g2)
    e_ref[...] = ilv(e1, e2)
    split = cwl + lam * wf
    bnd = ilv(split, cw[1:, :])
    big = jnp.full_like(split[0:1, :], 3.4e38)
    bnd_ref[...] = jnp.concatenate([bnd[:-1, :], big], axis=0)


def _prep_tables(uw, uh, ud, ul):
    uwT = uw.T
    uhT = uh.T
    udT = jnp.pad(ud.T, ((0, 1), (0, 0)))
    ulT = ul.T
    shp = jax.ShapeDtypeStruct((NSUB, D), jnp.float32)
    return pl.pallas_call(
        _prep_kernel,
        out_shape=[shp] * 5,
    )(uwT, uhT, udT, ulT)


def _tc_main_kernel(x_ref, a_ref, b_ref, g_ref, e_ref, bnd_ref, o_ref):
    x = x_ref[...]
    xc = jnp.clip(x, -BOUND, BOUND)
    shape = x.shape
    bc = lambda r: jnp.broadcast_to(r, shape)
    a = bc(a_ref[0:1, :])
    b = bc(b_ref[0:1, :])
    g = bc(g_ref[0:1, :])
    e = bc(e_ref[0:1, :])
    for j in range(1, NSUB):
        m = xc >= bnd_ref[j - 1 : j, :]
        a = jnp.where(m, bc(a_ref[j : j + 1, :]), a)
        b = jnp.where(m, bc(b_ref[j : j + 1, :]), b)
        g = jnp.where(m, bc(g_ref[j : j + 1, :]), g)
        e = jnp.where(m, bc(e_ref[j : j + 1, :]), e)
    out = (a + b * xc) / (g + e * xc)
    inside = (x >= -BOUND) & (x <= BOUND)
    o_ref[...] = jnp.where(inside, out, x)


def _tc_main(x2, a, b, g, e, bnd):
    n = x2.shape[0]
    br = 256
    tab_spec = pl.BlockSpec((NSUB, D), lambda i: (0, 0))
    return pl.pallas_call(
        _tc_main_kernel,
        grid=(n // br,),
        in_specs=[
            pl.BlockSpec((br, D), lambda i: (i, 0)),
            tab_spec, tab_spec, tab_spec, tab_spec, tab_spec,
        ],
        out_specs=pl.BlockSpec((br, D), lambda i: (i, 0)),
        out_shape=jax.ShapeDtypeStruct((n, D), jnp.float32),
    )(x2, a, b, g, e, bnd)


_NW = 32
_NSTRIPE = 16
_CPT = D // _NSTRIPE
_SC_R = 128


def _sc_compute_chunk(xb, ob, tabs, lanes):
    a_v, b_v, g_v, e_v, bnd_v = tabs

    @plsc.parallel_loop(0, _SC_R, 1, unroll=2)
    def _row(r):
        for q in range(_CPT // 16):
            dbase = (lanes + q * 16) * 16
            xv = xb[r, pl.ds(q * 16, 16)]
            xcv = jnp.clip(xv, -BOUND, BOUND)
            lo = jnp.zeros((16,), jnp.int32)
            for step in (8, 4, 2, 1):
                mid = lo + step
                bv = plsc.load_gather(bnd_v, [dbase + mid])
                lo = jnp.where(xcv >= bv, mid, lo)
            idx = dbase + lo
            av = plsc.load_gather(a_v, [idx])
            bv2 = plsc.load_gather(b_v, [idx])
            gv = plsc.load_gather(g_v, [idx])
            ev = plsc.load_gather(e_v, [idx])
            out = (av + bv2 * xcv) / (gv + ev * xcv)
            inside = (xv >= -BOUND) & (xv <= BOUND)
            ob[r, pl.ds(q * 16, 16)] = jnp.where(inside, out, xv)


def _sc_main_body(x_hbm, a_hbm, b_hbm, g_hbm, e_hbm, bnd_hbm, out_hbm,
                  a_v, b_v, g_v, e_v, bnd_v, xb0, xb1, ob0, ob1,
                  si0, si1, so0, so1):
    n = x_hbm.shape[0]
    half_rows = n // 2
    nch = half_rows // _SC_R
    cid = lax.axis_index("c")
    sid = lax.axis_index("s")
    w = sid * 2 + cid
    stripe = w % _NSTRIPE
    half = w // _NSTRIPE
    c0 = stripe * _CPT
    r_base = half * half_rows

    for hbm, v in ((a_hbm, a_v), (b_hbm, b_v), (g_hbm, g_v), (e_hbm, e_v),
                   (bnd_hbm, bnd_v)):
        pltpu.sync_copy(hbm.at[pl.ds(c0 * 16, _CPT * 16)], v)

    lanes = lax.iota(jnp.int32, 16)
    tabs = (a_v, b_v, g_v, e_v, bnd_v)
    xbufs = (xb0, xb1)
    obufs = (ob0, ob1)
    isems = (si0, si1)
    osems = (so0, so1)

    def xslice(cc):
        return x_hbm.at[pl.ds(r_base + cc * _SC_R, _SC_R), pl.ds(c0, _CPT)]

    def oslice(cc):
        return out_hbm.at[pl.ds(r_base + cc * _SC_R, _SC_R), pl.ds(c0, _CPT)]

    pltpu.async_copy(xslice(0), xb0, si0)
    pltpu.async_copy(xslice(1), xb1, si1)

    @pl.loop(0, nch, step=2)
    def _chunks(gg):
        for bsel in range(2):
            cc = gg + bsel
            xb, ob = xbufs[bsel], obufs[bsel]
            isem, osem = isems[bsel], osems[bsel]
            pltpu.make_async_copy(xslice(cc), xb, isem).wait()

            @pl.when(cc >= 2)
            def _():
                pltpu.make_async_copy(ob, oslice(cc - 2), osem).wait()

            _sc_compute_chunk(xb, ob, tabs, lanes)

            @pl.when(cc + 2 < nch)
            def _():
                pltpu.async_copy(xslice(cc + 2), xb, isem)

            pltpu.async_copy(ob, oslice(cc), osem)

    pltpu.make_async_copy(ob0, oslice(nch - 2), so0).wait()
    pltpu.make_async_copy(ob1, oslice(nch - 1), so1).wait()


def _sc_main(x2, a, b, g, e, bnd):
    a_sc = a.T.reshape(-1)
    b_sc = b.T.reshape(-1)
    g_sc = g.T.reshape(-1)
    e_sc = e.T.reshape(-1)
    bnd_sc = jnp.concatenate([jnp.zeros((D, 1), jnp.float32), bnd.T[:, :15]],
                             axis=1).reshape(-1)
    n = x2.shape[0]
    mesh = plsc.VectorSubcoreMesh(core_axis_name="c", subcore_axis_name="s")
    f = functools.partial(
        pl.kernel,
        out_type=jax.ShapeDtypeStruct((n, D), jnp.float32),
        mesh=mesh,
        compiler_params=pltpu.CompilerParams(needs_layout_passes=False),
        scratch_types=[
            pltpu.VMEM((_CPT * 16,), jnp.float32),
            pltpu.VMEM((_CPT * 16,), jnp.float32),
            pltpu.VMEM((_CPT * 16,), jnp.float32),
            pltpu.VMEM((_CPT * 16,), jnp.float32),
            pltpu.VMEM((_CPT * 16,), jnp.float32),
            pltpu.VMEM((_SC_R, _CPT), jnp.float32),
            pltpu.VMEM((_SC_R, _CPT), jnp.float32),
            pltpu.VMEM((_SC_R, _CPT), jnp.float32),
            pltpu.VMEM((_SC_R, _CPT), jnp.float32),
            pltpu.SemaphoreType.DMA,
            pltpu.SemaphoreType.DMA,
            pltpu.SemaphoreType.DMA,
            pltpu.SemaphoreType.DMA,
        ],
    )(_sc_main_body)
    return f(x2, a_sc, b_sc, g_sc, e_sc, bnd_sc)


def kernel(x, unnormalized_widths, unnormalized_heights, unnormalized_derivatives,
           unnormalized_lambdas):
    a, b, g, e, bnd = _prep_tables(unnormalized_widths, unnormalized_heights,
                                   unnormalized_derivatives, unnormalized_lambdas)
    n = x.shape[0] * x.shape[1]
    x2 = x.reshape(n, D)
    out = _sc_main(x2, a, b, g, e, bnd)
    return out.reshape(x.shape)


# speedup vs baseline: 2.5476x; 2.5476x over previous
import functools
import jax
import jax.numpy as jnp
from jax import lax
from jax.experimental import pallas as pl
from jax.experimental.pallas import tpu as pltpu
from jax.experimental.pallas import tpu_sc as plsc

D = 2048
K = 8
BOUND = 3.0
MIN_BW = 1e-3
MIN_BH = 1e-3
MIN_D = 1e-3
MIN_L = 0.025

NSUB = 2 * K


def _softmax0(v):
    m = jnp.max(v, axis=0, keepdims=True)
    e = jnp.exp(v - m)
    return e / jnp.sum(e, axis=0, keepdims=True)


def _knot_rows(frac):
    acc = frac[0:1, :]
    cs = [acc]
    for k in range(1, K):
        acc = acc + frac[k : k + 1, :]
        cs.append(acc)
    ones = jnp.ones_like(frac[0:1, :])
    rows = [jnp.full_like(frac[0:1, :], -BOUND)]
    for k in range(K - 1):
        rows.append(2.0 * BOUND * cs[k] - BOUND)
    rows.append(BOUND * ones)
    kn = jnp.concatenate(rows, axis=0)
    lengths = kn[1:, :] - kn[:-1, :]
    return lengths, kn


def _prep_kernel(uw_ref, uh_ref, ud_ref, ul_ref, a_ref, b_ref, g_ref, e_ref, bnd_ref):
    uw = uw_ref[...]
    uh = uh_ref[...]
    ud = ud_ref[0 : K - 1, :]
    ul = ul_ref[...]

    w = MIN_BW + (1.0 - MIN_BW * K) * _softmax0(uw)
    h = MIN_BH + (1.0 - MIN_BH * K) * _softmax0(uh)
    wf, cw = _knot_rows(w)
    hf, ch = _knot_rows(h)

    sp = jnp.maximum(ud, 0.0) + jnp.log(1.0 + jnp.exp(-jnp.abs(ud)))
    dmid = MIN_D + sp
    dend = jnp.full_like(dmid[0:1, :], 1.0 - MIN_D)
    dfull = jnp.concatenate([dend, dmid, dend], axis=0)

    lam = (1.0 - 2.0 * MIN_L) / (1.0 + jnp.exp(-ul)) + MIN_L

    d0 = dfull[:-1, :]
    d1 = dfull[1:, :]
    delta = hf / wf
    wb = jnp.sqrt(d0 / d1)
    wc = (lam * d0 + (1.0 - lam) * wb * d1) / delta
    ya = ch[:-1, :]
    yb = ch[:-1, :] + hf
    yc = ((1.0 - lam) * ya + lam * wb * yb) / ((1.0 - lam) + lam * wb)
    iw = 1.0 / wf
    cwl = cw[:-1, :]
    t0 = -cwl * iw
    wcyc = wc * yc
    wbyb = wb * yb

    a1 = ya * lam + t0 * (wcyc - ya)
    b1 = iw * (wcyc - ya)
    g1 = lam + t0 * (wc - 1.0)
    e1 = iw * (wc - 1.0)
    a2 = wcyc - lam * wbyb + t0 * (wbyb - wcyc)
    b2 = iw * (wbyb - wcyc)
    g2 = wc - lam * wb + t0 * (wb - wc)
    e2 = iw * (wb - wc)

    def ilv(p, q):
        rows = []
        for k in range(K):
            rows.append(p[k : k + 1, :])
            rows.append(q[k : k + 1, :])
        return jnp.concatenate(rows, axis=0)

    a_ref[...] = ilv(a1, a2)
    b_ref[...] = ilv(b1, b2)
    g_ref[...] = ilv(g1, g2)
    e_ref[...] = ilv(e1, e2)
    split = cwl + lam * wf
    bnd = ilv(split, cw[1:, :])
    big = jnp.full_like(split[0:1, :], 3.4e38)
    bnd_ref[...] = jnp.concatenate([bnd[:-1, :], big], axis=0)


def _prep_tables(uw, uh, ud, ul):
    uwT = uw.T
    uhT = uh.T
    udT = jnp.pad(ud.T, ((0, 1), (0, 0)))
    ulT = ul.T
    shp = jax.ShapeDtypeStruct((NSUB, D), jnp.float32)
    return pl.pallas_call(
        _prep_kernel,
        out_shape=[shp] * 5,
    )(uwT, uhT, udT, ulT)


def _tc_main_kernel(x_ref, a_ref, b_ref, g_ref, e_ref, bnd_ref, o_ref):
    x = x_ref[...]
    xc = jnp.clip(x, -BOUND, BOUND)
    shape = x.shape
    bc = lambda r: jnp.broadcast_to(r, shape)
    a = bc(a_ref[0:1, :])
    b = bc(b_ref[0:1, :])
    g = bc(g_ref[0:1, :])
    e = bc(e_ref[0:1, :])
    for j in range(1, NSUB):
        m = xc >= bnd_ref[j - 1 : j, :]
        a = jnp.where(m, bc(a_ref[j : j + 1, :]), a)
        b = jnp.where(m, bc(b_ref[j : j + 1, :]), b)
        g = jnp.where(m, bc(g_ref[j : j + 1, :]), g)
        e = jnp.where(m, bc(e_ref[j : j + 1, :]), e)
    out = (a + b * xc) / (g + e * xc)
    inside = (x >= -BOUND) & (x <= BOUND)
    o_ref[...] = jnp.where(inside, out, x)


def _tc_main(x2, a, b, g, e, bnd, row0=0):
    n = x2.shape[0]
    br = 256
    ob = row0 // br
    tab_spec = pl.BlockSpec((NSUB, D), lambda i: (0, 0))
    return pl.pallas_call(
        _tc_main_kernel,
        grid=(n // br - ob,),
        in_specs=[
            pl.BlockSpec((br, D), lambda i: (i + ob, 0)),
            tab_spec, tab_spec, tab_spec, tab_spec, tab_spec,
        ],
        out_specs=pl.BlockSpec((br, D), lambda i: (i + ob, 0)),
        out_shape=jax.ShapeDtypeStruct((n, D), jnp.float32),
    )(x2, a, b, g, e, bnd)


_NW = 32
_NSTRIPE = 16
_CPT = D // _NSTRIPE
_SC_R = 128


def _sc_compute_chunk(xb, ob, tabs, lanes):
    a_v, b_v, g_v, e_v, bnd_v = tabs

    @plsc.parallel_loop(0, _SC_R, 1, unroll=2)
    def _row(r):
        for q in range(_CPT // 16):
            dbase = (lanes + q * 16) * 16
            xv = xb[r, pl.ds(q * 16, 16)]
            xcv = jnp.clip(xv, -BOUND, BOUND)
            lo = jnp.zeros((16,), jnp.int32)
            for step in (8, 4, 2, 1):
                mid = lo + step
                bv = plsc.load_gather(bnd_v, [dbase + mid])
                lo = jnp.where(xcv >= bv, mid, lo)
            idx = dbase + lo
            av = plsc.load_gather(a_v, [idx])
            bv2 = plsc.load_gather(b_v, [idx])
            gv = plsc.load_gather(g_v, [idx])
            ev = plsc.load_gather(e_v, [idx])
            out = (av + bv2 * xcv) / (gv + ev * xcv)
            ob[r, pl.ds(q * 16, 16)] = jnp.where(xcv == xv, out, xv)


def _sc_main_body(x_hbm, a_hbm, b_hbm, g_hbm, e_hbm, bnd_hbm, out_hbm,
                  a_v, b_v, g_v, e_v, bnd_v, xb0, xb1, ob0, ob1,
                  si0, si1, so0, so1):
    n = out_hbm.shape[0]
    half_rows = n // 2
    nch = half_rows // _SC_R
    cid = lax.axis_index("c")
    sid = lax.axis_index("s")
    w = sid * 2 + cid
    stripe = w % _NSTRIPE
    half = w // _NSTRIPE
    c0 = stripe * _CPT
    r_base = half * half_rows

    for hbm, v in ((a_hbm, a_v), (b_hbm, b_v), (g_hbm, g_v), (e_hbm, e_v),
                   (bnd_hbm, bnd_v)):
        pltpu.sync_copy(hbm.at[pl.ds(c0 * 16, _CPT * 16)], v)

    lanes = lax.iota(jnp.int32, 16)
    tabs = (a_v, b_v, g_v, e_v, bnd_v)
    xbufs = (xb0, xb1)
    obufs = (ob0, ob1)
    isems = (si0, si1)
    osems = (so0, so1)

    def xslice(cc):
        return x_hbm.at[pl.ds(r_base + cc * _SC_R, _SC_R), pl.ds(c0, _CPT)]

    def oslice(cc):
        return out_hbm.at[pl.ds(r_base + cc * _SC_R, _SC_R), pl.ds(c0, _CPT)]

    pltpu.async_copy(xslice(0), xb0, si0)
    pltpu.async_copy(xslice(1), xb1, si1)

    @pl.loop(0, nch, step=2)
    def _chunks(gg):
        for bsel in range(2):
            cc = gg + bsel
            xb, ob = xbufs[bsel], obufs[bsel]
            isem, osem = isems[bsel], osems[bsel]
            pltpu.make_async_copy(xslice(cc), xb, isem).wait()

            @pl.when(cc >= 2)
            def _():
                pltpu.make_async_copy(ob, oslice(cc - 2), osem).wait()

            _sc_compute_chunk(xb, ob, tabs, lanes)

            @pl.when(cc + 2 < nch)
            def _():
                pltpu.async_copy(xslice(cc + 2), xb, isem)

            pltpu.async_copy(ob, oslice(cc), osem)

    pltpu.make_async_copy(ob0, oslice(nch - 2), so0).wait()
    pltpu.make_async_copy(ob1, oslice(nch - 1), so1).wait()


def _sc_main(x2, a, b, g, e, bnd, n_sc=None):
    a_sc = a.T.reshape(-1)
    b_sc = b.T.reshape(-1)
    g_sc = g.T.reshape(-1)
    e_sc = e.T.reshape(-1)
    bnd_sc = jnp.concatenate([jnp.zeros((D, 1), jnp.float32), bnd.T[:, :15]],
                             axis=1).reshape(-1)
    n = x2.shape[0] if n_sc is None else n_sc
    mesh = plsc.VectorSubcoreMesh(core_axis_name="c", subcore_axis_name="s")
    f = functools.partial(
        pl.kernel,
        out_type=jax.ShapeDtypeStruct((n, D), jnp.float32),
        mesh=mesh,
        compiler_params=pltpu.CompilerParams(needs_layout_passes=False),
        scratch_types=[
            pltpu.VMEM((_CPT * 16,), jnp.float32),
            pltpu.VMEM((_CPT * 16,), jnp.float32),
            pltpu.VMEM((_CPT * 16,), jnp.float32),
            pltpu.VMEM((_CPT * 16,), jnp.float32),
            pltpu.VMEM((_CPT * 16,), jnp.float32),
            pltpu.VMEM((_SC_R, _CPT), jnp.float32),
            pltpu.VMEM((_SC_R, _CPT), jnp.float32),
            pltpu.VMEM((_SC_R, _CPT), jnp.float32),
            pltpu.VMEM((_SC_R, _CPT), jnp.float32),
            pltpu.SemaphoreType.DMA,
            pltpu.SemaphoreType.DMA,
            pltpu.SemaphoreType.DMA,
            pltpu.SemaphoreType.DMA,
        ],
    )(_sc_main_body)
    return f(x2, a_sc, b_sc, g_sc, e_sc, bnd_sc)


def kernel(x, unnormalized_widths, unnormalized_heights, unnormalized_derivatives,
           unnormalized_lambdas):
    a, b, g, e, bnd = _prep_tables(unnormalized_widths, unnormalized_heights,
                                   unnormalized_derivatives, unnormalized_lambdas)
    n = x.shape[0] * x.shape[1]
    x2 = x.reshape(n, D)
    n_sc = 5632
    sc_part = _sc_main(x2, a, b, g, e, bnd, n_sc)
    tc_full = _tc_main(x2, a, b, g, e, bnd, row0=n_sc)
    out = lax.dynamic_update_slice(tc_full, sc_part, (0, 0))
    return out.reshape(x.shape)


# speedup vs baseline: 2.8144x; 1.1047x over previous
import functools
import jax
import jax.numpy as jnp
from jax import lax
from jax.experimental import pallas as pl
from jax.experimental.pallas import tpu as pltpu
from jax.experimental.pallas import tpu_sc as plsc

D = 2048
K = 8
BOUND = 3.0
MIN_BW = 1e-3
MIN_BH = 1e-3
MIN_D = 1e-3
MIN_L = 0.025

NSUB = 2 * K


def _softmax0(v):
    m = jnp.max(v, axis=0, keepdims=True)
    e = jnp.exp(v - m)
    return e / jnp.sum(e, axis=0, keepdims=True)


def _knot_rows(frac):
    acc = frac[0:1, :]
    cs = [acc]
    for k in range(1, K):
        acc = acc + frac[k : k + 1, :]
        cs.append(acc)
    ones = jnp.ones_like(frac[0:1, :])
    rows = [jnp.full_like(frac[0:1, :], -BOUND)]
    for k in range(K - 1):
        rows.append(2.0 * BOUND * cs[k] - BOUND)
    rows.append(BOUND * ones)
    kn = jnp.concatenate(rows, axis=0)
    lengths = kn[1:, :] - kn[:-1, :]
    return lengths, kn


def _prep_kernel(uw_ref, uh_ref, ud_ref, ul_ref, a_ref, b_ref, g_ref, e_ref, bnd_ref,
                 at_ref, bt_ref, gt_ref, et_ref, bndt_ref):
    uw = uw_ref[...]
    uh = uh_ref[...]
    ud = ud_ref[0 : K - 1, :]
    ul = ul_ref[...]

    w = MIN_BW + (1.0 - MIN_BW * K) * _softmax0(uw)
    h = MIN_BH + (1.0 - MIN_BH * K) * _softmax0(uh)
    wf, cw = _knot_rows(w)
    hf, ch = _knot_rows(h)

    sp = jnp.maximum(ud, 0.0) + jnp.log(1.0 + jnp.exp(-jnp.abs(ud)))
    dmid = MIN_D + sp
    dend = jnp.full_like(dmid[0:1, :], 1.0 - MIN_D)
    dfull = jnp.concatenate([dend, dmid, dend], axis=0)

    lam = (1.0 - 2.0 * MIN_L) / (1.0 + jnp.exp(-ul)) + MIN_L

    d0 = dfull[:-1, :]
    d1 = dfull[1:, :]
    delta = hf / wf
    wb = jnp.sqrt(d0 / d1)
    wc = (lam * d0 + (1.0 - lam) * wb * d1) / delta
    ya = ch[:-1, :]
    yb = ch[:-1, :] + hf
    yc = ((1.0 - lam) * ya + lam * wb * yb) / ((1.0 - lam) + lam * wb)
    iw = 1.0 / wf
    cwl = cw[:-1, :]
    t0 = -cwl * iw
    wcyc = wc * yc
    wbyb = wb * yb

    a1 = ya * lam + t0 * (wcyc - ya)
    b1 = iw * (wcyc - ya)
    g1 = lam + t0 * (wc - 1.0)
    e1 = iw * (wc - 1.0)
    a2 = wcyc - lam * wbyb + t0 * (wbyb - wcyc)
    b2 = iw * (wbyb - wcyc)
    g2 = wc - lam * wb + t0 * (wb - wc)
    e2 = iw * (wb - wc)

    def ilv(p, q):
        rows = []
        for k in range(K):
            rows.append(p[k : k + 1, :])
            rows.append(q[k : k + 1, :])
        return jnp.concatenate(rows, axis=0)

    av = ilv(a1, a2)
    bv = ilv(b1, b2)
    gv = ilv(g1, g2)
    ev = ilv(e1, e2)
    a_ref[...] = av
    b_ref[...] = bv
    g_ref[...] = gv
    e_ref[...] = ev
    split = cwl + lam * wf
    bnd = ilv(split, cw[1:, :])
    big = jnp.full_like(split[0:1, :], 3.4e38)
    bnd_ref[...] = jnp.concatenate([bnd[:-1, :], big], axis=0)
    at_ref[...] = av.T
    bt_ref[...] = bv.T
    gt_ref[...] = gv.T
    et_ref[...] = ev.T
    bnd_sh = jnp.concatenate([big, bnd[:-1, :]], axis=0)
    bndt_ref[...] = bnd_sh.T


def _prep_tables(uw, uh, ud, ul):
    uwT = uw.T
    uhT = uh.T
    udT = jnp.pad(ud.T, ((0, 1), (0, 0)))
    ulT = ul.T
    shp = jax.ShapeDtypeStruct((NSUB, D), jnp.float32)
    shp_t = jax.ShapeDtypeStruct((D, NSUB), jnp.float32)
    return pl.pallas_call(
        _prep_kernel,
        out_shape=[shp] * 5 + [shp_t] * 5,
    )(uwT, uhT, udT, ulT)


def _tc_main_kernel(x_ref, a_ref, b_ref, g_ref, e_ref, bnd_ref, o_ref):
    x = x_ref[...]
    xc = jnp.clip(x, -BOUND, BOUND)
    shape = x.shape
    bc = lambda r: jnp.broadcast_to(r, shape)
    a = bc(a_ref[0:1, :])
    b = bc(b_ref[0:1, :])
    g = bc(g_ref[0:1, :])
    e = bc(e_ref[0:1, :])
    for j in range(1, NSUB):
        m = xc >= bnd_ref[j - 1 : j, :]
        a = jnp.where(m, bc(a_ref[j : j + 1, :]), a)
        b = jnp.where(m, bc(b_ref[j : j + 1, :]), b)
        g = jnp.where(m, bc(g_ref[j : j + 1, :]), g)
        e = jnp.where(m, bc(e_ref[j : j + 1, :]), e)
    out = (a + b * xc) / (g + e * xc)
    inside = (x >= -BOUND) & (x <= BOUND)
    o_ref[...] = jnp.where(inside, out, x)


def _tc_main(x2, a, b, g, e, bnd, row0=0):
    n = x2.shape[0]
    br = 256
    ob = row0 // br
    tab_spec = pl.BlockSpec((NSUB, D), lambda i: (0, 0))
    return pl.pallas_call(
        _tc_main_kernel,
        grid=(n // br - ob,),
        in_specs=[
            pl.BlockSpec((br, D), lambda i: (i + ob, 0)),
            tab_spec, tab_spec, tab_spec, tab_spec, tab_spec,
        ],
        out_specs=pl.BlockSpec((br, D), lambda i: (i + ob, 0)),
        out_shape=jax.ShapeDtypeStruct((n, D), jnp.float32),
    )(x2, a, b, g, e, bnd)


_NSTRIPE = 16
_CPT = D // _NSTRIPE
_SC_R = 128


def _sc_compute_chunk(xb, ob, tabs, lanes):
    a_v, b_v, g_v, e_v, bnd_v = tabs

    for q in range(_CPT // 16):
        dbase = (lanes + q * 16) * 16
        bv8 = plsc.load_gather(bnd_v, [dbase + 8])
        bv4 = plsc.load_gather(bnd_v, [dbase + 4])
        bv12 = plsc.load_gather(bnd_v, [dbase + 12])

        @plsc.parallel_loop(0, _SC_R, 1, unroll=2)
        def _row(r):
            xv = xb[r, pl.ds(q * 16, 16)]
            xcv = jnp.clip(xv, -BOUND, BOUND)
            m1 = xcv >= bv8
            lo = jnp.where(m1, 8, 0)
            bl2 = jnp.where(m1, bv12, bv4)
            lo = jnp.where(xcv >= bl2, lo + 4, lo)
            for step in (2, 1):
                mid = lo + step
                bv = plsc.load_gather(bnd_v, [dbase + mid])
                lo = jnp.where(xcv >= bv, mid, lo)
            idx = dbase + lo
            av = plsc.load_gather(a_v, [idx])
            bv2 = plsc.load_gather(b_v, [idx])
            gv = plsc.load_gather(g_v, [idx])
            ev = plsc.load_gather(e_v, [idx])
            out = (av + bv2 * xcv) / (gv + ev * xcv)
            ob[r, pl.ds(q * 16, 16)] = jnp.where(xcv == xv, out, xv)


def _sc_main_body(x_hbm, a_hbm, b_hbm, g_hbm, e_hbm, bnd_hbm, out_hbm,
                  a_v, b_v, g_v, e_v, bnd_v, xb0, xb1, ob0, ob1,
                  si0, si1, so0, so1):
    n = out_hbm.shape[0]
    half_rows = n // 2
    nch = half_rows // _SC_R
    cid = lax.axis_index("c")
    sid = lax.axis_index("s")
    w = sid * 2 + cid
    stripe = w % _NSTRIPE
    half = w // _NSTRIPE
    c0 = stripe * _CPT
    r_base = half * half_rows

    for hbm, v in ((a_hbm, a_v), (b_hbm, b_v), (g_hbm, g_v), (e_hbm, e_v),
                   (bnd_hbm, bnd_v)):
        pltpu.sync_copy(hbm.at[pl.ds(c0 * 16, _CPT * 16)], v)

    lanes = lax.iota(jnp.int32, 16)
    tabs = (a_v, b_v, g_v, e_v, bnd_v)
    xbufs = (xb0, xb1)
    obufs = (ob0, ob1)
    isems = (si0, si1)
    osems = (so0, so1)

    def xslice(cc):
        return x_hbm.at[pl.ds(r_base + cc * _SC_R, _SC_R), pl.ds(c0, _CPT)]

    def oslice(cc):
        return out_hbm.at[pl.ds(r_base + cc * _SC_R, _SC_R), pl.ds(c0, _CPT)]

    pltpu.async_copy(xslice(0), xb0, si0)
    pltpu.async_copy(xslice(1), xb1, si1)

    @pl.loop(0, nch, step=2)
    def _chunks(gg):
        for bsel in range(2):
            cc = gg + bsel
            xb, ob = xbufs[bsel], obufs[bsel]
            isem, osem = isems[bsel], osems[bsel]
            pltpu.make_async_copy(xslice(cc), xb, isem).wait()

            @pl.when(cc >= 2)
            def _():
                pltpu.make_async_copy(ob, oslice(cc - 2), osem).wait()

            _sc_compute_chunk(xb, ob, tabs, lanes)

            @pl.when(cc + 2 < nch)
            def _():
                pltpu.async_copy(xslice(cc + 2), xb, isem)

            pltpu.async_copy(ob, oslice(cc), osem)

    pltpu.make_async_copy(ob0, oslice(nch - 2), so0).wait()
    pltpu.make_async_copy(ob1, oslice(nch - 1), so1).wait()


def _sc_main(x2, at, bt, gt, et, bndt, n_sc=None):
    a_sc = at.reshape(-1)
    b_sc = bt.reshape(-1)
    g_sc = gt.reshape(-1)
    e_sc = et.reshape(-1)
    bnd_sc = bndt.reshape(-1)
    n = x2.shape[0] if n_sc is None else n_sc
    mesh = plsc.VectorSubcoreMesh(core_axis_name="c", subcore_axis_name="s")
    f = functools.partial(
        pl.kernel,
        out_type=jax.ShapeDtypeStruct((n, D), jnp.float32),
        mesh=mesh,
        compiler_params=pltpu.CompilerParams(needs_layout_passes=False),
        scratch_types=[
            pltpu.VMEM((_CPT * 16,), jnp.float32),
            pltpu.VMEM((_CPT * 16,), jnp.float32),
            pltpu.VMEM((_CPT * 16,), jnp.float32),
            pltpu.VMEM((_CPT * 16,), jnp.float32),
            pltpu.VMEM((_CPT * 16,), jnp.float32),
            pltpu.VMEM((_SC_R, _CPT), jnp.float32),
            pltpu.VMEM((_SC_R, _CPT), jnp.float32),
            pltpu.VMEM((_SC_R, _CPT), jnp.float32),
            pltpu.VMEM((_SC_R, _CPT), jnp.float32),
            pltpu.SemaphoreType.DMA,
            pltpu.SemaphoreType.DMA,
            pltpu.SemaphoreType.DMA,
            pltpu.SemaphoreType.DMA,
        ],
    )(_sc_main_body)
    return f(x2, a_sc, b_sc, g_sc, e_sc, bnd_sc)


def kernel(x, unnormalized_widths, unnormalized_heights, unnormalized_derivatives,
           unnormalized_lambdas):
    a, b, g, e, bnd, at, bt, gt, et, bndt = _prep_tables(
        unnormalized_widths, unnormalized_heights,
        unnormalized_derivatives, unnormalized_lambdas)
    n = x.shape[0] * x.shape[1]
    x2 = x.reshape(n, D)
    n_sc = 7168
    sc_part = _sc_main(x2, at, bt, gt, et, bndt, n_sc)
    tc_full = _tc_main(x2, a, b, g, e, bnd, row0=n_sc)
    out = lax.dynamic_update_slice(tc_full, sc_part, (0, 0))
    return out.reshape(x.shape)
